# Initial kernel scaffold; baseline (speedup 1.0000x reference)
import jax
import jax.numpy as jnp
from jax.experimental import pallas as pl


def kernel(inp, gn):
    raise NotImplementedError("write your pallas kernel here")


# speedup vs baseline: 13.1034x; 13.1034x over previous
import functools

import jax
import jax.numpy as jnp
from jax.experimental import pallas as pl
from jax.experimental.pallas import tpu as pltpu

_BS = 8
_ROWS = 32
_N = 4096
_K = 16


def _topk_onehot_kernel(inp_ref, gn_ref, out_ref):
    x = inp_ref[...] + gn_ref[0]
    cur = x
    m = None
    for _ in range(_K):
        m = jnp.max(cur, axis=-1, keepdims=True)
        cur = jnp.where(cur >= m, -jnp.inf, cur)
    mask = x >= m
    rank = mask.astype(jnp.float32)
    sh = 1
    while sh < _N:
        z = jnp.zeros((_ROWS, sh), jnp.float32)
        rank = rank + jnp.concatenate([z, rank[:, :-sh]], axis=-1)
        sh *= 2
    jv = jax.lax.broadcasted_iota(jnp.int32, (1, _K, 1), 1).astype(jnp.float32) + 1.0
    hard = jnp.where(mask[:, None, :] & (rank[:, None, :] == jv), 1.0, 0.0)
    out_ref[0] = hard


@jax.jit
def kernel(inp, gn):
    out = pl.pallas_call(
        _topk_onehot_kernel,
        grid=(_BS,),
        in_specs=[
            pl.BlockSpec((_ROWS, _N), lambda b: (0, 0)),
            pl.BlockSpec((1, _ROWS, _N), lambda b: (b, 0, 0)),
        ],
        out_specs=pl.BlockSpec((1, _ROWS, _K, _N), lambda b: (b, 0, 0, 0)),
        out_shape=jax.ShapeDtypeStruct((_BS, _ROWS, _K, _N), jnp.float32),
    )(inp, gn)
    return out


# speedup vs baseline: 18.1430x; 1.3846x over previous
import jax
import jax.numpy as jnp
from jax.experimental import pallas as pl

_BS = 8
_ROWS = 32
_N = 4096
_K = 16


def _batcher_pairs(n):
    pairs = []

    def merge(lo, length, r):
        step = r * 2
        if step < length:
            merge(lo, length, step)
            merge(lo + r, length, step)
            for i in range(lo + r, lo + length - r, step):
                pairs.append((i, i + r))
        else:
            pairs.append((lo, lo + r))

    def sort(lo, length):
        if length > 1:
            m = length // 2
            sort(lo, m)
            sort(lo + m, m)
            merge(lo, length, 1)

    sort(0, n)
    return pairs


_SORT_PAIRS = _batcher_pairs(_K)


def _topk_onehot_kernel(inp_ref, gn_ref, out_ref):
    x = inp_ref[...] + gn_ref[0]
    iota_n = jax.lax.broadcasted_iota(jnp.int32, (_ROWS, _N), 1)
    cur = x
    idx_cols = []
    for i in range(_K):
        m = jnp.max(cur, axis=-1, keepdims=True)
        eq = cur == m
        idx_cols.append(jnp.min(jnp.where(eq, iota_n, _N), axis=-1, keepdims=True))
        if i + 1 < _K:
            cur = jnp.where(eq, -jnp.inf, cur)
    for a, b in _SORT_PAIRS:
        lo = jnp.minimum(idx_cols[a], idx_cols[b])
        hi = jnp.maximum(idx_cols[a], idx_cols[b])
        idx_cols[a], idx_cols[b] = lo, hi
    sorted_idx = jnp.concatenate([c[:, None, :] for c in idx_cols], axis=1)
    hard = jnp.where(
        jax.lax.broadcasted_iota(jnp.int32, (1, 1, _N), 2) == sorted_idx,
        1.0,
        0.0,
    )
    out_ref[0] = hard


@jax.jit
def kernel(inp, gn):
    out = pl.pallas_call(
        _topk_onehot_kernel,
        grid=(_BS,),
        in_specs=[
            pl.BlockSpec((_ROWS, _N), lambda b: (0, 0)),
            pl.BlockSpec((1, _ROWS, _N), lambda b: (b, 0, 0)),
        ],
        out_specs=pl.BlockSpec((1, _ROWS, _K, _N), lambda b: (b, 0, 0, 0)),
        out_shape=jax.ShapeDtypeStruct((_BS, _ROWS, _K, _N), jnp.float32),
    )(inp, gn)
    return out


# speedup vs baseline: 18.9456x; 1.0442x over previous
import functools

import jax
import jax.numpy as jnp
from jax.experimental import pallas as pl

_BS = 8
_ROWS = 32
_N = 4096
_K = 16
_NCOL = _N // 128
_BB = 2


def _batcher_pairs(n):
    pairs = []

    def merge(lo, length, r):
        step = r * 2
        if step < length:
            merge(lo, length, step)
            merge(lo + r, length, step)
            for i in range(lo + r, lo + length - r, step):
                pairs.append((i, i + r))
        else:
            pairs.append((lo, lo + r))

    def sort(lo, length):
        if length > 1:
            m = length // 2
            sort(lo, m)
            sort(lo + m, m)
            merge(lo, length, 1)

    sort(0, n)
    return pairs


_SORT_PAIRS = _batcher_pairs(_K)


def _sort16(cols, descending):
    cols = list(cols)
    for a, b in _SORT_PAIRS:
        hi = jnp.maximum(cols[a], cols[b])
        lo = jnp.minimum(cols[a], cols[b])
        if descending:
            cols[a], cols[b] = hi, lo
        else:
            cols[a], cols[b] = lo, hi
    return cols


def _lane_top16(cols, descending):
    a = _sort16(cols[:_K], descending)
    b = _sort16(cols[_K:], descending)
    if descending:
        return [jnp.maximum(a[i], b[_K - 1 - i]) for i in range(_K)]
    return [jnp.minimum(a[i], b[_K - 1 - i]) for i in range(_K)]


def _topk_onehot_kernel(inp_ref, gn_ref, out_ref):
    inp = inp_ref[...]
    gn = gn_ref[...].reshape(_BB * _ROWS, _N)
    x = jnp.concatenate([inp] * _BB, axis=0) + gn
    cols = [x[:, i * 128 : (i + 1) * 128] for i in range(_NCOL)]
    cand = jnp.concatenate(_lane_top16(cols, descending=True), axis=-1)
    m = None
    for i in range(_K):
        m = jnp.max(cand, axis=-1, keepdims=True)
        if i + 1 < _K:
            cand = jnp.where(cand == m, -jnp.inf, cand)
    iota_f = jax.lax.broadcasted_iota(jnp.int32, (_BB * _ROWS, _N), 1).astype(
        jnp.float32
    )
    y = jnp.where(x >= m, iota_f, float(2 * _N))
    ycols = [y[:, i * 128 : (i + 1) * 128] for i in range(_NCOL)]
    ycand = jnp.concatenate(_lane_top16(ycols, descending=False), axis=-1)
    idx_cols = []
    for i in range(_K):
        mi = jnp.min(ycand, axis=-1, keepdims=True)
        idx_cols.append(mi)
        if i + 1 < _K:
            ycand = jnp.where(ycand == mi, float(2 * _N), ycand)
    sorted_idx = jnp.concatenate([c[:, None, :] for c in idx_cols], axis=1)
    hard = jnp.where(
        jax.lax.broadcasted_iota(jnp.int32, (1, 1, _N), 2).astype(jnp.float32)
        == sorted_idx,
        1.0,
        0.0,
    )
    out_ref[...] = hard.reshape(_BB, _ROWS, _K, _N)


@jax.jit
def kernel(inp, gn):
    out = pl.pallas_call(
        _topk_onehot_kernel,
        grid=(_BS // _BB,),
        in_specs=[
            pl.BlockSpec((_ROWS, _N), lambda b: (0, 0)),
            pl.BlockSpec((_BB, _ROWS, _N), lambda b: (b, 0, 0)),
        ],
        out_specs=pl.BlockSpec((_BB, _ROWS, _K, _N), lambda b: (b, 0, 0, 0)),
        out_shape=jax.ShapeDtypeStruct((_BS, _ROWS, _K, _N), jnp.float32),
    )(inp, gn)
    return out
